# Initial kernel scaffold; baseline (speedup 1.0000x reference)
import functools

import jax
import jax.numpy as jnp
from jax import lax
from jax.experimental import pallas as pl
from jax.experimental.pallas import tpu as pltpu
from jax.experimental.pallas import tpu_sc as plsc

N = 10000
D = 128
NC = 2
NS = 16
LANES = 16


def _seg_sum_sc(table, src, dst, ea):
    E = src.shape[0]
    NW = NC * NS
    assert E % NW == 0
    EPW = E // NW
    K = 80
    assert EPW % K == 0
    NCHUNK = EPW // K
    RPT = N // NS
    assert N % NS == 0
    ZB = 80
    weighted = ea is not None

    mesh = plsc.VectorSubcoreMesh(
        core_axis_name="c", subcore_axis_name="s",
        num_cores=NC, num_subcores=NS)

    scratch = [
        pltpu.VMEM_SHARED((N, D), jnp.float32),
        pltpu.VMEM((K,), jnp.int32),
        pltpu.VMEM((K,), jnp.int32),
        pltpu.VMEM((K, D), jnp.float32),
        pltpu.VMEM((K,), jnp.float32),
        pltpu.SemaphoreType.DMA,
    ]

    @functools.partial(
        pl.kernel,
        out_type=jax.ShapeDtypeStruct((NC, N, D), jnp.float32),
        mesh=mesh,
        scratch_types=scratch,
    )
    def seg_kernel(*refs):
        if weighted:
            (table_h, src_h, dst_h, ea_h, out_h,
             acc, si, di, rows, eab, sem) = refs
        else:
            (table_h, src_h, dst_h, out_h,
             acc, si, di, rows, eab, sem) = refs
        c = lax.axis_index("c")
        s = lax.axis_index("s")
        w = c * NS + s

        def zrow(i, carry):
            for j in range(D // LANES):
                rows[i, pl.ds(j * LANES, LANES)] = jnp.zeros(
                    (LANES,), jnp.float32)
            return carry
        lax.fori_loop(0, ZB, zrow, 0)
        nfull = RPT // ZB
        rem = RPT - nfull * ZB
        for t in range(nfull):
            pltpu.sync_copy(rows.at[pl.ds(0, ZB)],
                            acc.at[pl.ds(s * RPT + t * ZB, ZB)])
        if rem:
            pltpu.sync_copy(rows.at[pl.ds(0, rem)],
                            acc.at[pl.ds(s * RPT + nfull * ZB, rem)])
        plsc.subcore_barrier()

        base0 = w * EPW

        def chunk(g, carry):
            base = base0 + g * K
            pltpu.sync_copy(src_h.at[pl.ds(base, K)], si)
            pltpu.sync_copy(dst_h.at[pl.ds(base, K)], di)
            pltpu.async_copy(table_h.at[si], rows, sem).wait()
            if weighted:
                pltpu.sync_copy(ea_h.at[pl.ds(base, K)], eab)

                def scale(k2, carry2):
                    a = eab[k2]
                    av = lax.broadcast(a, (LANES,))
                    for j in range(D // LANES):
                        sl = pl.ds(j * LANES, LANES)
                        rows[k2, sl] = rows[k2, sl] * av
                    return carry2
                lax.fori_loop(0, K, scale, 0)
            pltpu.sync_copy(rows, acc.at[di], add=True)
            return carry
        lax.fori_loop(0, NCHUNK, chunk, 0)
        plsc.subcore_barrier()

        for t in range(nfull):
            off = s * RPT + t * ZB
            pltpu.sync_copy(acc.at[pl.ds(off, ZB)],
                            out_h.at[c, pl.ds(off, ZB)])
        if rem:
            off = s * RPT + nfull * ZB
            pltpu.sync_copy(acc.at[pl.ds(off, rem)],
                            out_h.at[c, pl.ds(off, rem)])

    if weighted:
        return seg_kernel(table, src, dst, ea)
    return seg_kernel(table, src, dst)


def _gn(pre, w, b, ms):
    mean = jnp.mean(pre, axis=0, keepdims=True)
    cen = pre - ms * mean
    var = jnp.mean(cen * cen, axis=0, keepdims=True)
    return w * (cen * lax.rsqrt(var + 1e-5)) + b


def _dense1_body(parts, x, w1, b1, gw, gb, gms, o):
    agg = parts[0] + parts[1] + x[...]
    pre = lax.dot_general(agg, w1[...], (((1,), (1,)), ((), ())),
                          preferred_element_type=jnp.float32) + b1[...]
    o[...] = jnp.maximum(_gn(pre, gw[...], gb[...], gms[...]), 0.0)


def _dense2_body(parts, h_ref, w2, b2, gw, gb, gms,
                 wih1, bi1, bh1, wih2, bi2, bh2,
                 xcat_o, h1_o, c1_o, h2_o, c2_o):
    h = h_ref[...]
    agg = parts[0] + parts[1] + h
    pre = lax.dot_general(agg, w2[...], (((1,), (1,)), ((), ())),
                          preferred_element_type=jnp.float32) + b2[...]
    h2 = jnp.maximum(_gn(pre, gw[...], gb[...], gms[...]), 0.0)
    xcat = jnp.concatenate([h, h2], axis=1)
    xcat_o[...] = xcat

    g = lax.dot_general(xcat, wih1[...], (((1,), (1,)), ((), ())),
                        preferred_element_type=jnp.float32)
    g = g + bi1[...] + bh1[...]
    i1 = jax.nn.sigmoid(g[:, 0:D])
    gg1 = jnp.tanh(g[:, 2 * D:3 * D])
    o1 = jax.nn.sigmoid(g[:, 3 * D:4 * D])
    c1 = i1 * gg1
    h1 = o1 * jnp.tanh(c1)
    h1_o[...] = h1
    c1_o[...] = c1

    g2 = lax.dot_general(h1, wih2[...], (((1,), (1,)), ((), ())),
                         preferred_element_type=jnp.float32)
    g2 = g2 + bi2[...] + bh2[...]
    i2 = jax.nn.sigmoid(g2[:, 0:D])
    gg2 = jnp.tanh(g2[:, 2 * D:3 * D])
    o2 = jax.nn.sigmoid(g2[:, 3 * D:4 * D])
    c2 = i2 * gg2
    h2n = o2 * jnp.tanh(c2)
    h2_o[...] = h2n
    c2_o[...] = c2


def kernel(x, edge_index, edge_attr, W1, b1, W2, b2,
           gn1_w, gn1_b, gn1_ms, gn2_w, gn2_b, gn2_ms,
           l1_wih, l1_whh, l1_bih, l1_bhh,
           l2_wih, l2_whh, l2_bih, l2_bhh):
    src = edge_index[0]
    dst = edge_index[1]
    row = lambda v: v.reshape(1, -1)

    parts1 = _seg_sum_sc(x, src, dst, None)
    h = pl.pallas_call(
        _dense1_body,
        out_shape=jax.ShapeDtypeStruct((N, D), jnp.float32),
    )(parts1, x, W1, row(b1), row(gn1_w), row(gn1_b), row(gn1_ms))

    parts2 = _seg_sum_sc(h, src, dst, edge_attr)
    outs = pl.pallas_call(
        _dense2_body,
        out_shape=(
            jax.ShapeDtypeStruct((N, 2 * D), jnp.float32),
            jax.ShapeDtypeStruct((N, D), jnp.float32),
            jax.ShapeDtypeStruct((N, D), jnp.float32),
            jax.ShapeDtypeStruct((N, D), jnp.float32),
            jax.ShapeDtypeStruct((N, D), jnp.float32),
        ),
    )(parts2, h, W2, row(b2), row(gn2_w), row(gn2_b), row(gn2_ms),
      l1_wih, row(l1_bih), row(l1_bhh), l2_wih, row(l2_bih), row(l2_bhh))
    return outs


# speedup vs baseline: 4.3232x; 4.3232x over previous
import functools

import jax
import jax.numpy as jnp
from jax import lax
from jax.experimental import pallas as pl
from jax.experimental.pallas import tpu as pltpu
from jax.experimental.pallas import tpu_sc as plsc

N = 10000
D = 128
NC = 2
NS = 16
LANES = 16


def _seg_sum_sc(table, src, dst, ea):
    E = src.shape[0]
    NW = NC * NS
    assert E % NW == 0
    EPW = E // NW
    K = 80
    assert EPW % K == 0
    NCHUNK = EPW // K
    RPT = 624
    REM = N - NS * RPT
    assert 0 <= REM < RPT and REM % 8 == 0
    ZB = 80
    weighted = ea is not None

    mesh = plsc.VectorSubcoreMesh(
        core_axis_name="c", subcore_axis_name="s",
        num_cores=NC, num_subcores=NS)

    scratch = [
        pltpu.VMEM_SHARED((N, D), jnp.float32),
        pltpu.VMEM((K,), jnp.int32),
        pltpu.VMEM((K,), jnp.int32),
        pltpu.VMEM((K, D), jnp.float32),
        pltpu.VMEM((K,), jnp.float32),
        pltpu.SemaphoreType.DMA,
    ]

    @functools.partial(
        pl.kernel,
        out_type=jax.ShapeDtypeStruct((NC, N, D), jnp.float32),
        mesh=mesh,
        scratch_types=scratch,
    )
    def seg_kernel(*refs):
        if weighted:
            (table_h, src_h, dst_h, ea_h, out_h,
             acc, si, di, rows, eab, sem) = refs
        else:
            (table_h, src_h, dst_h, out_h,
             acc, si, di, rows, eab, sem) = refs
        c = lax.axis_index("c")
        s = lax.axis_index("s")
        w = c * NS + s

        def zrow(i, carry):
            for j in range(D // LANES):
                rows[i, pl.ds(j * LANES, LANES)] = jnp.zeros(
                    (LANES,), jnp.float32)
            return carry
        lax.fori_loop(0, ZB, zrow, 0)
        nfull = RPT // ZB
        rem = RPT - nfull * ZB
        for t in range(nfull):
            pltpu.sync_copy(rows.at[pl.ds(0, ZB)],
                            acc.at[pl.ds(s * RPT + t * ZB, ZB)])
        if rem:
            pltpu.sync_copy(rows.at[pl.ds(0, rem)],
                            acc.at[pl.ds(s * RPT + nfull * ZB, rem)])

        @pl.when(s == 0)
        def _():
            if REM:
                pltpu.sync_copy(rows.at[pl.ds(0, REM)],
                                acc.at[pl.ds(NS * RPT, REM)])
        plsc.subcore_barrier()

        base0 = w * EPW

        def chunk(g, carry):
            base = base0 + g * K
            pltpu.sync_copy(src_h.at[pl.ds(base, K)], si)
            pltpu.sync_copy(dst_h.at[pl.ds(base, K)], di)
            pltpu.async_copy(table_h.at[si], rows, sem).wait()
            if weighted:
                pltpu.sync_copy(ea_h.at[pl.ds(base, K)], eab)

                def scale(gi, carry2):
                    av16 = eab[pl.ds(gi * LANES, LANES)]
                    for l in range(LANES):
                        av = lax.broadcast(av16[l], (LANES,))
                        ri = gi * LANES + l
                        for j in range(D // LANES):
                            sl = pl.ds(j * LANES, LANES)
                            rows[ri, sl] = rows[ri, sl] * av
                    return carry2
                lax.fori_loop(0, K // LANES, scale, 0)
            pltpu.sync_copy(rows, acc.at[di], add=True)
            return carry
        lax.fori_loop(0, NCHUNK, chunk, 0)
        plsc.subcore_barrier()

        for t in range(nfull):
            off = s * RPT + t * ZB
            pltpu.sync_copy(acc.at[pl.ds(off, ZB)],
                            out_h.at[c, pl.ds(off, ZB)])
        if rem:
            off = s * RPT + nfull * ZB
            pltpu.sync_copy(acc.at[pl.ds(off, rem)],
                            out_h.at[c, pl.ds(off, rem)])

        @pl.when(s == 0)
        def _():
            if REM:
                pltpu.sync_copy(acc.at[pl.ds(NS * RPT, REM)],
                                out_h.at[c, pl.ds(NS * RPT, REM)])

    if weighted:
        return seg_kernel(table, src, dst, ea)
    return seg_kernel(table, src, dst)


def _gn(pre, w, b, ms):
    mean = jnp.mean(pre, axis=0, keepdims=True)
    cen = pre - ms * mean
    var = jnp.mean(cen * cen, axis=0, keepdims=True)
    return w * (cen * lax.rsqrt(var + 1e-5)) + b


def _dense1_body(parts, x, w1, b1, gw, gb, gms, o):
    agg = parts[0] + parts[1] + x[...]
    pre = lax.dot_general(agg, w1[...], (((1,), (1,)), ((), ())),
                          preferred_element_type=jnp.float32) + b1[...]
    o[...] = jnp.maximum(_gn(pre, gw[...], gb[...], gms[...]), 0.0)


def _dense2_body(parts, h_ref, w2, b2, gw, gb, gms,
                 wih1, bi1, bh1, wih2, bi2, bh2,
                 xcat_o, h1_o, c1_o, h2_o, c2_o):
    h = h_ref[...]
    agg = parts[0] + parts[1] + h
    pre = lax.dot_general(agg, w2[...], (((1,), (1,)), ((), ())),
                          preferred_element_type=jnp.float32) + b2[...]
    h2 = jnp.maximum(_gn(pre, gw[...], gb[...], gms[...]), 0.0)
    xcat = jnp.concatenate([h, h2], axis=1)
    xcat_o[...] = xcat

    g = lax.dot_general(xcat, wih1[...], (((1,), (1,)), ((), ())),
                        preferred_element_type=jnp.float32)
    g = g + bi1[...] + bh1[...]
    i1 = jax.nn.sigmoid(g[:, 0:D])
    gg1 = jnp.tanh(g[:, 2 * D:3 * D])
    o1 = jax.nn.sigmoid(g[:, 3 * D:4 * D])
    c1 = i1 * gg1
    h1 = o1 * jnp.tanh(c1)
    h1_o[...] = h1
    c1_o[...] = c1

    g2 = lax.dot_general(h1, wih2[...], (((1,), (1,)), ((), ())),
                         preferred_element_type=jnp.float32)
    g2 = g2 + bi2[...] + bh2[...]
    i2 = jax.nn.sigmoid(g2[:, 0:D])
    gg2 = jnp.tanh(g2[:, 2 * D:3 * D])
    o2 = jax.nn.sigmoid(g2[:, 3 * D:4 * D])
    c2 = i2 * gg2
    h2n = o2 * jnp.tanh(c2)
    h2_o[...] = h2n
    c2_o[...] = c2


def kernel(x, edge_index, edge_attr, W1, b1, W2, b2,
           gn1_w, gn1_b, gn1_ms, gn2_w, gn2_b, gn2_ms,
           l1_wih, l1_whh, l1_bih, l1_bhh,
           l2_wih, l2_whh, l2_bih, l2_bhh):
    src = edge_index[0]
    dst = edge_index[1]
    row = lambda v: v.reshape(1, -1)

    parts1 = _seg_sum_sc(x, src, dst, None)
    h = pl.pallas_call(
        _dense1_body,
        out_shape=jax.ShapeDtypeStruct((N, D), jnp.float32),
    )(parts1, x, W1, row(b1), row(gn1_w), row(gn1_b), row(gn1_ms))

    parts2 = _seg_sum_sc(h, src, dst, edge_attr)
    outs = pl.pallas_call(
        _dense2_body,
        out_shape=(
            jax.ShapeDtypeStruct((N, 2 * D), jnp.float32),
            jax.ShapeDtypeStruct((N, D), jnp.float32),
            jax.ShapeDtypeStruct((N, D), jnp.float32),
            jax.ShapeDtypeStruct((N, D), jnp.float32),
            jax.ShapeDtypeStruct((N, D), jnp.float32),
        ),
    )(parts2, h, W2, row(b2), row(gn2_w), row(gn2_b), row(gn2_ms),
      l1_wih, row(l1_bih), row(l1_bhh), l2_wih, row(l2_bih), row(l2_bhh))
    return outs
